# baseline (speedup 1.0000x reference)
import jax
import jax.numpy as jnp
from jax.experimental import pallas as pl
from jax.experimental.pallas import tpu as pltpu

B = 768
S = 8
D = 768
E = 8
K = 2
HID = 1536
EPS = 1e-6
CAP = 3.0

LBLK = 128
NJ = D // LBLK


def _erf(v):
    a1, a2, a3, a4, a5, pp = (
        0.254829592,
        -0.284496736,
        1.421413741,
        -1.453152027,
        1.061405429,
        0.3275911,
    )
    sgn = jnp.sign(v)
    av = jnp.abs(v)
    t = 1.0 / (1.0 + pp * av)
    y = 1.0 - (((((a5 * t + a4) * t) + a3) * t + a2) * t + a1) * t * jnp.exp(
        -av * av
    )
    return sgn * y


def _gelu_exact(z):
    return 0.5 * z * (1.0 + _erf(z * 0.7071067811865476))


def _moe_kernel(
    x_ref,
    wg_ref,
    bg_ref,
    w1_ref,
    b1_ref,
    w2_ref,
    b2_ref,
    out_ref,
    h_scr,
    logits_scr,
    scale_scr,
):
    p = pl.program_id(0)
    j = pl.program_id(1)

    @pl.when(p == 0)
    def _pass0():
        xb = x_ref[...]
        x2 = xb.reshape(B, S * LBLK)
        h = jnp.dot(w1_ref[...], x2, preferred_element_type=jnp.float32)
        h_scr[j] = _gelu_exact(h + b1_ref[...]).astype(jnp.bfloat16)
        xr = xb.reshape(B * S, LBLK)
        lg = jax.lax.dot_general(
            xr,
            wg_ref[...],
            dimension_numbers=(((1,), (1,)), ((), ())),
            preferred_element_type=jnp.float32,
        )

        @pl.when(j == 0)
        def _():
            logits_scr[...] = lg + bg_ref[...]

        @pl.when(j > 0)
        def _():
            logits_scr[...] = logits_scr[...] + lg

    @pl.when(p == 1)
    def _pass1():
        @pl.when(j == 0)
        def _gate():
            logits = logits_scr[...]
            m = jnp.max(logits, axis=-1, keepdims=True)
            ex = jnp.exp(logits - m)
            prob = ex / jnp.sum(ex, axis=-1, keepdims=True)
            e_iota = jax.lax.broadcasted_iota(jnp.int32, prob.shape, 1)
            top1i = jnp.argmax(prob, axis=-1).astype(jnp.int32)
            pm = jnp.where(e_iota == top1i[:, None], -jnp.inf, prob)
            top2i = jnp.argmax(pm, axis=-1).astype(jnp.int32)
            t1 = top1i.reshape(B, S)
            t2 = top2i.reshape(B, S)
            si = jax.lax.broadcasted_iota(jnp.int32, (B, S, S), 2)
            mask1 = jnp.any(t1[:, :, None] == si, axis=1)
            mask2 = jnp.any(t2[:, :, None] == si, axis=1)
            pr = prob.reshape(B, S, E)
            m0 = pr[:, :, 0] * mask1.astype(jnp.float32)
            m1 = pr[:, :, 1] * mask2.astype(jnp.float32)
            d0 = jnp.sum(m0, axis=0, keepdims=True) + EPS
            d1 = jnp.sum(m1, axis=0, keepdims=True) + EPS
            scale_scr[...] = (m0 / d0 + m1 / d1) * CAP

        h = h_scr[j]
        o = jnp.dot(w2_ref[...], h, preferred_element_type=jnp.float32)
        o = o + b2_ref[...]
        o3 = o.reshape(D, S, LBLK) * scale_scr[...][:, :, None]
        out_ref[...] = o3


@jax.jit
def kernel(x, Wg, bg, W1, b1, W2, b2):
    out = pl.pallas_call(
        _moe_kernel,
        grid=(2, NJ),
        in_specs=[
            pl.BlockSpec((B, S, LBLK), lambda p, j: (0, 0, jnp.where(p == 0, j, NJ - 1))),
            pl.BlockSpec((E, LBLK), lambda p, j: (0, jnp.where(p == 0, j, NJ - 1))),
            pl.BlockSpec((1, E), lambda p, j: (0, 0)),
            pl.BlockSpec((HID, B), lambda p, j: (0, 0)),
            pl.BlockSpec((HID, 1), lambda p, j: (0, 0)),
            pl.BlockSpec((D, HID), lambda p, j: (0, 0)),
            pl.BlockSpec((D, 1), lambda p, j: (0, 0)),
        ],
        out_specs=pl.BlockSpec(
            (D, S, LBLK), lambda p, j: (0, 0, jnp.where(p == 0, 0, j))
        ),
        out_shape=jax.ShapeDtypeStruct((D, S, D), jnp.float32),
        scratch_shapes=[
            pltpu.VMEM((NJ, HID, S * LBLK), jnp.bfloat16),
            pltpu.VMEM((B * S, E), jnp.float32),
            pltpu.VMEM((B, S), jnp.float32),
        ],
    )(
        x,
        Wg,
        bg.reshape(1, E),
        W1,
        b1.reshape(HID, 1),
        W2.astype(jnp.bfloat16),
        b2.reshape(D, 1),
    )

    return out
